# baseline (speedup 1.0000x reference)
import jax
import jax.numpy as jnp
from jax import lax
from jax.experimental import pallas as pl

NUM_CLASSES = 1000
BATCH = 16384
ROWS = 512


def _onehot_block(x_ref, out_ref):
    x = x_ref[0, 0, :]
    cols = lax.broadcasted_iota(jnp.int32, (ROWS, NUM_CLASSES), 1)
    out_ref[...] = jnp.where(x[:, None] == cols, 1.0, 0.0).astype(jnp.float32)


def kernel(x):
    x = x.astype(jnp.int32)
    grid = BATCH // ROWS
    x3 = x.reshape(grid, 1, ROWS)
    return pl.pallas_call(
        _onehot_block,
        grid=(grid,),
        in_specs=[pl.BlockSpec((1, 1, ROWS), lambda i: (i, 0, 0))],
        out_specs=pl.BlockSpec((ROWS, NUM_CLASSES), lambda i: (i, 0)),
        out_shape=jax.ShapeDtypeStruct((BATCH, NUM_CLASSES), jnp.float32),
    )(x3)
